# baseline (speedup 1.0000x reference)
import functools

import jax
import jax.numpy as jnp
from jax import lax
from jax.experimental import pallas as pl
from jax.experimental.pallas import tpu as pltpu
from jax.experimental.pallas import tpu_sc as plsc

N_CORES = 2
N_SUBCORES = 16
CHUNK = 128
F = 128
FH = F // 2
CNT_W = 16
NBUF = 4


def _sc_aggregate(hst, src4, dst3, n_nodes, n_pad):
    n_chunks = dst3.shape[1]
    rows_per_tile = n_pad // N_SUBCORES
    zrows = rows_per_tile // 5

    mesh = plsc.VectorSubcoreMesh(core_axis_name="c", subcore_axis_name="s")

    @functools.partial(
        pl.kernel,
        out_type=[
            jax.ShapeDtypeStruct((N_CORES, n_pad, FH), jnp.float32),
            jax.ShapeDtypeStruct((N_CORES, n_pad, CNT_W), jnp.float32),
        ],
        mesh=mesh,
        scratch_types=[
            pltpu.VMEM((n_chunks, CHUNK), jnp.int32),
            pltpu.VMEM((n_chunks, CHUNK), jnp.int32),
            [pltpu.VMEM((CHUNK, FH), jnp.float32) for _ in range(NBUF)],
            pltpu.VMEM((CHUNK, CNT_W), jnp.float32),
            pltpu.VMEM((zrows, CNT_W), jnp.float32),
            pltpu.VMEM_SHARED((n_pad, FH), jnp.float32),
            pltpu.VMEM_SHARED((n_pad, CNT_W), jnp.float32),
            [pltpu.SemaphoreType.DMA for _ in range(NBUF)],
            [pltpu.SemaphoreType.DMA for _ in range(NBUF)],
            [pltpu.SemaphoreType.DMA for _ in range(2)],
        ],
        compiler_params=pltpu.CompilerParams(use_tc_tiling_on_sc=False),
    )
    def agg(hst_hbm, src_hbm, dst_hbm, acc_hbm, cnt_hbm,
            srcv, dstv, bufs, ones_v, zcnt_v,
            acc_sh, cnt_sh, gsem, ssem, osem):
        c = lax.axis_index("c")
        s = lax.axis_index("s")

        pltpu.sync_copy(src_hbm.at[c, s], srcv)
        pltpu.sync_copy(dst_hbm.at[s], dstv)

        @pl.loop(0, CHUNK)
        def _(i):
            ones_v[i, :] = jnp.full((CNT_W,), 1.0, jnp.float32)
            for j in range(FH // 16):
                bufs[0][i, pl.ds(j * 16, 16)] = jnp.zeros((16,), jnp.float32)
            zcnt_v[i % zrows, :] = jnp.zeros((CNT_W,), jnp.float32)

        for j in range(rows_per_tile // zrows):
            r0 = s * rows_per_tile + j * zrows
            pltpu.sync_copy(bufs[0], acc_sh.at[pl.ds(r0, zrows)])
            pltpu.sync_copy(zcnt_v, cnt_sh.at[pl.ds(r0, zrows)])
        plsc.subcore_barrier()

        def fire_g(i, j):
            pltpu.async_copy(hst_hbm.at[srcv.at[i]], bufs[j], gsem[j])

        def fire_s(i, j):
            pltpu.async_copy(bufs[j], acc_sh.at[dstv.at[i]], ssem[j], add=True)

        def fire_o(i, j):
            pltpu.async_copy(ones_v, cnt_sh.at[dstv.at[i]], osem[j // 2],
                             add=True)

        for j in range(2):
            fire_g(j, j)

        @pl.loop(0, n_chunks, step=NBUF)
        def _(i):
            @pl.when(i > 0)
            def _():
                for k in range(2):
                    kk = 2 * k + (c != 0)
                    pltpu.make_async_copy(
                        ones_v, cnt_sh.at[dstv.at[i - NBUF + kk]],
                        osem[k]).wait()

            for jj in range(NBUF):
                t = i + jj
                b2 = (jj + 2) % NBUF
                pltpu.make_async_copy(hst_hbm.at[srcv.at[t]], bufs[jj],
                                      gsem[jj]).wait()
                fire_s(t, jj)

                @pl.when(c == (jj % 2))
                def _(t=t, jj=jj):
                    fire_o(t, jj)

                @pl.when(t >= 2)
                def _(t=t, b2=b2):
                    pltpu.make_async_copy(bufs[b2], acc_sh.at[dstv.at[t - 2]],
                                          ssem[b2]).wait()

                @pl.when(t + 2 < n_chunks)
                def _(t=t, b2=b2):
                    fire_g(t + 2, b2)

        for t in (n_chunks - 2, n_chunks - 1):
            pltpu.make_async_copy(bufs[t % NBUF], acc_sh.at[dstv.at[t]],
                                  ssem[t % NBUF]).wait()
        for k in range(2):
            kk = 2 * k + (c != 0)
            pltpu.make_async_copy(
                ones_v, cnt_sh.at[dstv.at[n_chunks - NBUF + kk]],
                osem[k]).wait()

        plsc.subcore_barrier()

        r0 = s * rows_per_tile
        pltpu.sync_copy(acc_sh.at[pl.ds(r0, rows_per_tile)],
                        acc_hbm.at[c, pl.ds(r0, rows_per_tile)])
        pltpu.sync_copy(cnt_sh.at[pl.ds(r0, rows_per_tile)],
                        cnt_hbm.at[c, pl.ds(r0, rows_per_tile)])

    return agg(hst, src4, dst3)


def _tc_combine(h, acc, cnt, w1t, w2t, b2):
    n = h.shape[0]
    br = 1000
    grid = (n // br,)

    def body(h_ref, acc_ref, cnt_ref, w1_ref, w2_ref, b_ref, o_ref):
        a = jnp.concatenate([acc_ref[0], acc_ref[1]], axis=1)
        cn = cnt_ref[0, :, 0:1] + cnt_ref[1, :, 0:1]
        inv = 1.0 / jnp.maximum(cn, 1.0)
        hn = a * inv
        t1 = jnp.dot(h_ref[...], w1_ref[...], preferred_element_type=jnp.float32)
        t2 = jnp.dot(hn, w2_ref[...], preferred_element_type=jnp.float32)
        o_ref[...] = t1 + t2 + b_ref[...]

    return pl.pallas_call(
        body,
        grid=grid,
        in_specs=[
            pl.BlockSpec((br, F), lambda i: (i, 0)),
            pl.BlockSpec((N_CORES, br, FH), lambda i: (0, i, 0)),
            pl.BlockSpec((N_CORES, br, CNT_W), lambda i: (0, i, 0)),
            pl.BlockSpec((F, F), lambda i: (0, 0)),
            pl.BlockSpec((F, F), lambda i: (0, 0)),
            pl.BlockSpec((1, F), lambda i: (0, 0)),
        ],
        out_specs=pl.BlockSpec((br, F), lambda i: (i, 0)),
        out_shape=jax.ShapeDtypeStruct((n, F), jnp.float32),
    )(h, acc, cnt, w1t, w2t, b2)


def kernel(h, edge_index, W, b):
    n_nodes, f_in = h.shape
    n_edges = edge_index.shape[1]
    n_pad = ((n_nodes + 40 * N_SUBCORES - 1) // (40 * N_SUBCORES)) * 40 * N_SUBCORES

    per_tile = n_edges // N_SUBCORES
    n_chunks = -(-per_tile // CHUNK)
    n_chunks = ((n_chunks + NBUF - 1) // NBUF) * NBUF
    pad = n_chunks * CHUNK - per_tile

    src = edge_index[0].reshape(N_SUBCORES, per_tile)
    dst = edge_index[1].reshape(N_SUBCORES, per_tile)
    if pad:
        pad_src = jnp.zeros((N_SUBCORES, pad), jnp.int32)
        spread = n_pad - n_nodes
        lanes = (jnp.arange(N_SUBCORES, dtype=jnp.int32)[:, None] * 37
                 + jnp.arange(pad, dtype=jnp.int32)[None, :])
        pad_dst = n_nodes + lanes % spread
        src = jnp.concatenate([src, pad_src], axis=1)
        dst = jnp.concatenate([dst, pad_dst], axis=1)
    src3 = src.reshape(N_SUBCORES, n_chunks, CHUNK)
    dst3 = dst.reshape(N_SUBCORES, n_chunks, CHUNK)
    src4 = jnp.stack([src3, src3 + n_nodes])

    hst = jnp.concatenate([h[:, :FH], h[:, FH:]], axis=0)

    w1t = W[:, :f_in].T
    w2t = W[:, f_in:].T
    b2 = b.reshape(1, -1)
    acc, cnt = _sc_aggregate(hst, src4, dst3, n_nodes, n_pad)
    return _tc_combine(h, acc, cnt, w1t, w2t, b2)


# speedup vs baseline: 1.0627x; 1.0627x over previous
import functools

import jax
import jax.numpy as jnp
from jax import lax
from jax.experimental import pallas as pl
from jax.experimental.pallas import tpu as pltpu
from jax.experimental.pallas import tpu_sc as plsc

N_CORES = 2
N_SUBCORES = 16
CHUNK = 128
F = 128
FH = F // 2
CNT_W = 16
NBUF = 4


def _sc_aggregate(hst, src4, dst3, n_nodes, n_pad):
    n_chunks = dst3.shape[1]
    rows_per_tile = n_pad // N_SUBCORES
    zrows = rows_per_tile // 5

    mesh = plsc.VectorSubcoreMesh(core_axis_name="c", subcore_axis_name="s")

    @functools.partial(
        pl.kernel,
        out_type=[
            jax.ShapeDtypeStruct((N_CORES, n_pad, FH), jnp.float32),
            jax.ShapeDtypeStruct((N_CORES, n_pad, CNT_W), jnp.float32),
        ],
        mesh=mesh,
        scratch_types=[
            pltpu.VMEM((n_chunks, CHUNK), jnp.int32),
            pltpu.VMEM((n_chunks, CHUNK), jnp.int32),
            [pltpu.VMEM((CHUNK, FH), jnp.float32) for _ in range(NBUF)],
            pltpu.VMEM((CHUNK, CNT_W), jnp.float32),
            pltpu.VMEM((zrows, CNT_W), jnp.float32),
            pltpu.VMEM_SHARED((n_pad, FH), jnp.float32),
            pltpu.VMEM_SHARED((n_pad, CNT_W), jnp.float32),
            [pltpu.SemaphoreType.DMA for _ in range(NBUF)],
            [pltpu.SemaphoreType.DMA for _ in range(NBUF)],
            [pltpu.SemaphoreType.DMA for _ in range(2)],
        ],
        compiler_params=pltpu.CompilerParams(use_tc_tiling_on_sc=False),
    )
    def agg(hst_hbm, src_hbm, dst_hbm, acc_hbm, cnt_hbm,
            srcv, dstv, bufs, ones_v, zcnt_v,
            acc_sh, cnt_sh, gsem, ssem, osem):
        c = lax.axis_index("c")
        s = lax.axis_index("s")

        pltpu.sync_copy(src_hbm.at[c, s], srcv)
        pltpu.sync_copy(dst_hbm.at[s], dstv)

        @pl.loop(0, CHUNK)
        def _(i):
            ones_v[i, :] = jnp.full((CNT_W,), 1.0, jnp.float32)
            for j in range(FH // 16):
                bufs[0][i, pl.ds(j * 16, 16)] = jnp.zeros((16,), jnp.float32)
            zcnt_v[i % zrows, :] = jnp.zeros((CNT_W,), jnp.float32)

        for j in range(rows_per_tile // zrows):
            r0 = s * rows_per_tile + j * zrows
            pltpu.sync_copy(bufs[0], acc_sh.at[pl.ds(r0, zrows)])
            pltpu.sync_copy(zcnt_v, cnt_sh.at[pl.ds(r0, zrows)])
        plsc.subcore_barrier()

        def fire_g(i, j):
            pltpu.async_copy(hst_hbm.at[srcv.at[i]], bufs[j], gsem[j])

        def drain_g(i, j):
            pltpu.make_async_copy(hst_hbm.at[srcv.at[i]], bufs[j],
                                  gsem[j]).wait()

        for j in range(NBUF):
            fire_g(j, j)

        @pl.loop(0, n_chunks, step=NBUF)
        def _(i):
            for jj in range(NBUF):
                t = i + jj
                drain_g(t, jj)
                pltpu.sync_copy(bufs[jj], acc_sh.at[dstv.at[t]], add=True)

                @pl.when(c == (jj % 2))
                def _(t=t):
                    pltpu.sync_copy(ones_v, cnt_sh.at[dstv.at[t]], add=True)

                @pl.when(t + NBUF < n_chunks)
                def _(t=t, jj=jj):
                    fire_g(t + NBUF, jj)

        plsc.subcore_barrier()

        r0 = s * rows_per_tile
        pltpu.sync_copy(acc_sh.at[pl.ds(r0, rows_per_tile)],
                        acc_hbm.at[c, pl.ds(r0, rows_per_tile)])
        pltpu.sync_copy(cnt_sh.at[pl.ds(r0, rows_per_tile)],
                        cnt_hbm.at[c, pl.ds(r0, rows_per_tile)])

    return agg(hst, src4, dst3)


def _tc_combine(h, acc, cnt, w1t, w2t, b2):
    n = h.shape[0]
    br = 1000
    grid = (n // br,)

    def body(h_ref, acc_ref, cnt_ref, w1_ref, w2_ref, b_ref, o_ref):
        a = jnp.concatenate([acc_ref[0], acc_ref[1]], axis=1)
        cn = cnt_ref[0, :, 0:1] + cnt_ref[1, :, 0:1]
        inv = 1.0 / jnp.maximum(cn, 1.0)
        hn = a * inv
        t1 = jnp.dot(h_ref[...], w1_ref[...], preferred_element_type=jnp.float32)
        t2 = jnp.dot(hn, w2_ref[...], preferred_element_type=jnp.float32)
        o_ref[...] = t1 + t2 + b_ref[...]

    return pl.pallas_call(
        body,
        grid=grid,
        in_specs=[
            pl.BlockSpec((br, F), lambda i: (i, 0)),
            pl.BlockSpec((N_CORES, br, FH), lambda i: (0, i, 0)),
            pl.BlockSpec((N_CORES, br, CNT_W), lambda i: (0, i, 0)),
            pl.BlockSpec((F, F), lambda i: (0, 0)),
            pl.BlockSpec((F, F), lambda i: (0, 0)),
            pl.BlockSpec((1, F), lambda i: (0, 0)),
        ],
        out_specs=pl.BlockSpec((br, F), lambda i: (i, 0)),
        out_shape=jax.ShapeDtypeStruct((n, F), jnp.float32),
    )(h, acc, cnt, w1t, w2t, b2)


def kernel(h, edge_index, W, b):
    n_nodes, f_in = h.shape
    n_edges = edge_index.shape[1]
    n_pad = ((n_nodes + 40 * N_SUBCORES - 1) // (40 * N_SUBCORES)) * 40 * N_SUBCORES

    per_tile = n_edges // N_SUBCORES
    n_chunks = -(-per_tile // CHUNK)
    n_chunks = ((n_chunks + NBUF - 1) // NBUF) * NBUF
    pad = n_chunks * CHUNK - per_tile

    src = edge_index[0].reshape(N_SUBCORES, per_tile)
    dst = edge_index[1].reshape(N_SUBCORES, per_tile)
    if pad:
        pad_src = jnp.zeros((N_SUBCORES, pad), jnp.int32)
        spread = n_pad - n_nodes
        lanes = (jnp.arange(N_SUBCORES, dtype=jnp.int32)[:, None] * 37
                 + jnp.arange(pad, dtype=jnp.int32)[None, :])
        pad_dst = n_nodes + lanes % spread
        src = jnp.concatenate([src, pad_src], axis=1)
        dst = jnp.concatenate([dst, pad_dst], axis=1)
    src3 = src.reshape(N_SUBCORES, n_chunks, CHUNK)
    dst3 = dst.reshape(N_SUBCORES, n_chunks, CHUNK)
    src4 = jnp.stack([src3, src3 + n_nodes])

    hst = jnp.concatenate([h[:, :FH], h[:, FH:]], axis=0)

    w1t = W[:, :f_in].T
    w2t = W[:, f_in:].T
    b2 = b.reshape(1, -1)
    acc, cnt = _sc_aggregate(hst, src4, dst3, n_nodes, n_pad)
    return _tc_combine(h, acc, cnt, w1t, w2t, b2)


# speedup vs baseline: 1.1321x; 1.0652x over previous
import functools

import jax
import jax.numpy as jnp
from jax import lax
from jax.experimental import pallas as pl
from jax.experimental.pallas import tpu as pltpu
from jax.experimental.pallas import tpu_sc as plsc

N_CORES = 2
N_SUBCORES = 16
CHUNK = 128
F = 128
FH = F // 2
CNT_W = 16


def _sc_aggregate(h_lo, h_hi, src3, dst3, n_nodes, n_pad):
    n_chunks = src3.shape[1]
    rows_per_tile = n_pad // N_SUBCORES
    zrows = rows_per_tile // 5

    mesh = plsc.VectorSubcoreMesh(core_axis_name="c", subcore_axis_name="s")

    @functools.partial(
        pl.kernel,
        out_type=[
            jax.ShapeDtypeStruct((N_CORES, n_pad, FH), jnp.float32),
            jax.ShapeDtypeStruct((N_CORES, n_pad, CNT_W), jnp.float32),
        ],
        mesh=mesh,
        scratch_types=[
            pltpu.VMEM((n_chunks, CHUNK), jnp.int32),
            pltpu.VMEM((n_chunks, CHUNK), jnp.int32),
            pltpu.VMEM((2 * CHUNK, FH), jnp.float32),
            pltpu.VMEM((CHUNK, CNT_W), jnp.float32),
            pltpu.VMEM((zrows, FH), jnp.float32),
            pltpu.VMEM((zrows, CNT_W), jnp.float32),
            pltpu.VMEM_SHARED((n_pad, FH), jnp.float32),
            pltpu.VMEM_SHARED((n_pad, CNT_W), jnp.float32),
            pltpu.SemaphoreType.DMA,
        ],
        compiler_params=pltpu.CompilerParams(use_tc_tiling_on_sc=False),
    )
    def agg(hlo_hbm, hhi_hbm, src_hbm, dst_hbm, acc_hbm, cnt_hbm,
            srcv, dstv, rows01, ones_v, zrow_v, zcnt_v,
            acc_sh, cnt_sh, gsem):
        c = lax.axis_index("c")
        s = lax.axis_index("s")

        pltpu.sync_copy(src_hbm.at[s], srcv)
        pltpu.sync_copy(dst_hbm.at[s], dstv)

        @pl.loop(0, CHUNK)
        def _(i):
            ones_v[i, :] = jnp.full((CNT_W,), 1.0, jnp.float32)

        @pl.loop(0, zrows)
        def _(i):
            for j in range(FH // 16):
                zrow_v[i, pl.ds(j * 16, 16)] = jnp.zeros((16,), jnp.float32)
            zcnt_v[i, :] = jnp.zeros((CNT_W,), jnp.float32)

        for j in range(rows_per_tile // zrows):
            r0 = s * rows_per_tile + j * zrows
            pltpu.sync_copy(zrow_v, acc_sh.at[pl.ds(r0, zrows)])
            pltpu.sync_copy(zcnt_v, cnt_sh.at[pl.ds(r0, zrows)])
        plsc.subcore_barrier()

        half0 = rows01.at[pl.ds(0, CHUNK)]
        half1 = rows01.at[pl.ds(CHUNK, CHUNK)]

        def run(h_half_hbm, parity):
            def fire(i, half):
                pltpu.async_copy(h_half_hbm.at[srcv.at[i]], half, gsem)

            def drain_pair():
                pltpu.make_async_copy(h_half_hbm.at[pl.ds(0, 2 * CHUNK)],
                                      rows01, gsem).wait()

            def scat(i, half, count):
                pltpu.sync_copy(half, acc_sh.at[dstv.at[i]], add=True)
                if count:
                    pltpu.sync_copy(ones_v, cnt_sh.at[dstv.at[i]], add=True)

            fire(0, half0)
            fire(1, half1)

            @pl.loop(0, n_chunks - 2, step=2)
            def _(i):
                drain_pair()
                scat(i, half0, parity == 0)
                fire(i + 2, half0)
                scat(i + 1, half1, parity == 1)
                fire(i + 3, half1)

            drain_pair()
            scat(n_chunks - 2, half0, parity == 0)
            scat(n_chunks - 1, half1, parity == 1)

        @pl.when(c == 0)
        def _():
            run(hlo_hbm, 0)

        @pl.when(c == 1)
        def _():
            run(hhi_hbm, 1)

        plsc.subcore_barrier()

        r0 = s * rows_per_tile
        pltpu.sync_copy(acc_sh.at[pl.ds(r0, rows_per_tile)],
                        acc_hbm.at[c, pl.ds(r0, rows_per_tile)])
        pltpu.sync_copy(cnt_sh.at[pl.ds(r0, rows_per_tile)],
                        cnt_hbm.at[c, pl.ds(r0, rows_per_tile)])

    return agg(h_lo, h_hi, src3, dst3)


def _tc_combine(h, acc, cnt, w1t, w2t, b2):
    n = h.shape[0]
    br = 1000
    grid = (n // br,)

    def body(h_ref, acc_ref, cnt_ref, w1_ref, w2_ref, b_ref, o_ref):
        a = jnp.concatenate([acc_ref[0], acc_ref[1]], axis=1)
        cn = cnt_ref[0, :, 0:1] + cnt_ref[1, :, 0:1]
        inv = 1.0 / jnp.maximum(cn, 1.0)
        hn = a * inv
        t1 = jnp.dot(h_ref[...], w1_ref[...], preferred_element_type=jnp.float32)
        t2 = jnp.dot(hn, w2_ref[...], preferred_element_type=jnp.float32)
        o_ref[...] = t1 + t2 + b_ref[...]

    return pl.pallas_call(
        body,
        grid=grid,
        in_specs=[
            pl.BlockSpec((br, F), lambda i: (i, 0)),
            pl.BlockSpec((N_CORES, br, FH), lambda i: (0, i, 0)),
            pl.BlockSpec((N_CORES, br, CNT_W), lambda i: (0, i, 0)),
            pl.BlockSpec((F, F), lambda i: (0, 0)),
            pl.BlockSpec((F, F), lambda i: (0, 0)),
            pl.BlockSpec((1, F), lambda i: (0, 0)),
        ],
        out_specs=pl.BlockSpec((br, F), lambda i: (i, 0)),
        out_shape=jax.ShapeDtypeStruct((n, F), jnp.float32),
    )(h, acc, cnt, w1t, w2t, b2)


def kernel(h, edge_index, W, b):
    n_nodes, f_in = h.shape
    n_edges = edge_index.shape[1]
    n_pad = ((n_nodes + 40 * N_SUBCORES - 1) // (40 * N_SUBCORES)) * 40 * N_SUBCORES

    per_tile = n_edges // N_SUBCORES
    n_chunks = -(-per_tile // CHUNK)
    if n_chunks % 2:
        n_chunks += 1
    pad = n_chunks * CHUNK - per_tile

    src = edge_index[0].reshape(N_SUBCORES, per_tile)
    dst = edge_index[1].reshape(N_SUBCORES, per_tile)
    if pad:
        pad_src = jnp.zeros((N_SUBCORES, pad), jnp.int32)
        spread = n_pad - n_nodes
        lanes = (jnp.arange(N_SUBCORES, dtype=jnp.int32)[:, None] * 37
                 + jnp.arange(pad, dtype=jnp.int32)[None, :])
        pad_dst = n_nodes + lanes % spread
        src = jnp.concatenate([src, pad_src], axis=1)
        dst = jnp.concatenate([dst, pad_dst], axis=1)
    src3 = src.reshape(N_SUBCORES, n_chunks, CHUNK)
    dst3 = dst.reshape(N_SUBCORES, n_chunks, CHUNK)

    h_lo = h[:, :FH]
    h_hi = h[:, FH:]
    w1t = W[:, :f_in].T
    w2t = W[:, f_in:].T
    b2 = b.reshape(1, -1)
    acc, cnt = _sc_aggregate(h_lo, h_hi, src3, dst3, n_nodes, n_pad)
    return _tc_combine(h, acc, cnt, w1t, w2t, b2)
